# baseline (speedup 1.0000x reference)
import jax
import jax.numpy as jnp
from jax.experimental import pallas as pl

_BN = 2048
_CW = 128


_CG = 512
_RG = 64


def _finish_half(val, idx, big):
    m = jnp.min(val, axis=1, keepdims=True)
    cand = jnp.where(val == m, idx, big)
    return m, jnp.min(cand, axis=1, keepdims=True)


def _assign_kernel(x_ref, x2_ref, c_ref, c2_ref, out_ref):
    x = x_ref[...]
    c = c_ref[...]
    x2 = x2_ref[...]
    c2 = c2_ref[...]
    k = c.shape[0]
    h = k // 2
    bn = x.shape[0]
    ct = c.T

    big = jnp.int32(h)
    dots = [jnp.dot(x, 2.0 * ct[:, g * _CG:(g + 1) * _CG],
                    preferred_element_type=jnp.float32)
            for g in range(k // _CG)]

    for r0 in range(0, bn, _RG):
        rows = slice(r0, r0 + _RG)
        x2r = x2[rows]
        iota_cw = jax.lax.broadcasted_iota(jnp.int32, (_RG, _CW), 1)
        halves = []
        for half in range(2):
            val = None
            idx = None
            for g in range(half * (h // _CG), (half + 1) * (h // _CG)):
                dot2 = dots[g][rows]
                s = x2r + c2[:, g * _CG:(g + 1) * _CG]
                for cc in range(_CG // _CW):
                    sl = slice(cc * _CW, (cc + 1) * _CW)
                    chunk = s[:, sl] - dot2[:, sl]
                    off = jnp.int32((g % (h // _CG)) * _CG + cc * _CW)
                    if val is None:
                        val = chunk
                        idx = iota_cw
                    else:
                        lt = chunk < val
                        val = jnp.where(lt, chunk, val)
                        idx = jnp.where(lt, iota_cw + off, idx)
            halves.append((val, idx))

        ma, ia = _finish_half(halves[0][0], halves[0][1], big)
        mb, ib = _finish_half(halves[1][0], halves[1][1], big)

        dist_a = jnp.sqrt(jnp.maximum(ma, 0.0))
        dist_b = jnp.sqrt(jnp.maximum(mb, 0.0))
        dist_a_r = dist_a.astype(jnp.bfloat16).astype(jnp.float32)
        pick_a = dist_a_r <= dist_b
        out_ref[rows, :] = jnp.where(pick_a, ia, ib + jnp.int32(h))


def kernel(X, centers):
    n, d = X.shape
    k, _ = centers.shape
    x2 = jnp.sum(X * X, axis=1, keepdims=True)
    c2 = jnp.sum(centers * centers, axis=1)[None, :]
    xb = X.astype(jnp.bfloat16)
    cb = centers.astype(jnp.bfloat16)
    grid = (n // _BN,)
    out = pl.pallas_call(
        _assign_kernel,
        grid=grid,
        in_specs=[
            pl.BlockSpec((_BN, d), lambda i: (i, 0)),
            pl.BlockSpec((_BN, 1), lambda i: (i, 0)),
            pl.BlockSpec((k, d), lambda i: (0, 0)),
            pl.BlockSpec((1, k), lambda i: (0, 0)),
        ],
        out_specs=pl.BlockSpec((_BN, 1), lambda i: (i, 0)),
        out_shape=jax.ShapeDtypeStruct((n, 1), jnp.int32),
    )(xb, x2, cb, c2)
    return out.reshape(n)


# speedup vs baseline: 1.0090x; 1.0090x over previous
import jax
import jax.numpy as jnp
from jax.experimental import pallas as pl

_BN = 2048
_CW = 128


_CG = 512
_RG = 256


def _finish_half(val, idx, big):
    m = jnp.min(val, axis=1, keepdims=True)
    cand = jnp.where(val == m, idx, big)
    return m, jnp.min(cand, axis=1, keepdims=True)


def _assign_kernel(x_ref, x2_ref, c_ref, c2_ref, out_ref):
    x = x_ref[...]
    c = c_ref[...]
    x2 = x2_ref[...]
    c2 = c2_ref[...]
    k = c.shape[0]
    h = k // 2
    bn = x.shape[0]
    ct = c.T

    big = jnp.int32(h)
    dots = [jnp.dot(x, 2.0 * ct[:, g * _CG:(g + 1) * _CG],
                    preferred_element_type=jnp.float32)
            for g in range(k // _CG)]

    for r0 in range(0, bn, _RG):
        rows = slice(r0, r0 + _RG)
        x2r = x2[rows]
        iota_cw = jax.lax.broadcasted_iota(jnp.int32, (_RG, _CW), 1)
        halves = []
        for half in range(2):
            val = None
            idx = None
            for g in range(half * (h // _CG), (half + 1) * (h // _CG)):
                dot2 = dots[g][rows]
                s = x2r + c2[:, g * _CG:(g + 1) * _CG]
                for cc in range(_CG // _CW):
                    sl = slice(cc * _CW, (cc + 1) * _CW)
                    chunk = s[:, sl] - dot2[:, sl]
                    off = jnp.int32((g % (h // _CG)) * _CG + cc * _CW)
                    if val is None:
                        val = chunk
                        idx = iota_cw
                    else:
                        lt = chunk < val
                        val = jnp.where(lt, chunk, val)
                        idx = jnp.where(lt, iota_cw + off, idx)
            halves.append((val, idx))

        ma, ia = _finish_half(halves[0][0], halves[0][1], big)
        mb, ib = _finish_half(halves[1][0], halves[1][1], big)

        dist_a = jnp.sqrt(jnp.maximum(ma, 0.0))
        dist_b = jnp.sqrt(jnp.maximum(mb, 0.0))
        dist_a_r = dist_a.astype(jnp.bfloat16).astype(jnp.float32)
        pick_a = dist_a_r <= dist_b
        out_ref[rows, :] = jnp.where(pick_a, ia, ib + jnp.int32(h))


def kernel(X, centers):
    n, d = X.shape
    k, _ = centers.shape
    x2 = jnp.sum(X * X, axis=1, keepdims=True)
    c2 = jnp.sum(centers * centers, axis=1)[None, :]
    xb = X.astype(jnp.bfloat16)
    cb = centers.astype(jnp.bfloat16)
    grid = (n // _BN,)
    out = pl.pallas_call(
        _assign_kernel,
        grid=grid,
        in_specs=[
            pl.BlockSpec((_BN, d), lambda i: (i, 0)),
            pl.BlockSpec((_BN, 1), lambda i: (i, 0)),
            pl.BlockSpec((k, d), lambda i: (0, 0)),
            pl.BlockSpec((1, k), lambda i: (0, 0)),
        ],
        out_specs=pl.BlockSpec((_BN, 1), lambda i: (i, 0)),
        out_shape=jax.ShapeDtypeStruct((n, 1), jnp.int32),
    )(xb, x2, cb, c2)
    return out.reshape(n)
